# speedup vs baseline: 2.9736x; 2.9736x over previous
import functools

import jax
import jax.numpy as jnp
from jax import lax
from jax.experimental import pallas as pl
from jax.experimental.pallas import tpu as pltpu
from jax.experimental.pallas import tpu_sc as plsc

_VOCAB = 100000
_D = 128
_B = 4096 * 50

_info = plsc.get_sparse_core_info()
_NC = _info.num_cores
_NS = _info.num_subcores
_NW = _NC * _NS
_B_PER_W = _B // _NW
_CHUNK = 128
_N_CHUNKS = _B_PER_W // _CHUNK

_mesh = plsc.VectorSubcoreMesh(core_axis_name="c", subcore_axis_name="s")


@functools.partial(
    pl.kernel,
    mesh=_mesh,
    out_type=jax.ShapeDtypeStruct((_B, _D), jnp.float32),
    scratch_types=[
        pltpu.VMEM((_N_CHUNKS, _CHUNK), jnp.int32),
        pltpu.VMEM((_CHUNK, _D), jnp.float32),
        pltpu.SemaphoreType.DMA,
    ],
)
def _sc_gather(idx_hbm, table_hbm, out_hbm, idx_v, rows_v, sem):
    wid = lax.axis_index("s") * _NC + lax.axis_index("c")
    base = wid * _B_PER_W
    pltpu.sync_copy(idx_hbm.at[wid], idx_v)

    def body(j, carry):
        pltpu.async_copy(table_hbm.at[idx_v.at[j]], rows_v, sem).wait()
        pltpu.sync_copy(rows_v, out_hbm.at[pl.ds(base + j * _CHUNK, _CHUNK)])
        return carry

    lax.fori_loop(0, _N_CHUNKS, body, 0)


def kernel(x, embedding_weights):
    idx = x.reshape(_NW, _N_CHUNKS, _CHUNK)
    out = _sc_gather(idx, embedding_weights)
    return out.reshape(x.shape[0], x.shape[1], _D)


# speedup vs baseline: 3.3150x; 1.1148x over previous
import functools

import jax
import jax.numpy as jnp
from jax import lax
from jax.experimental import pallas as pl
from jax.experimental.pallas import tpu as pltpu
from jax.experimental.pallas import tpu_sc as plsc

_VOCAB = 100000
_D = 128
_B = 4096 * 50

_info = plsc.get_sparse_core_info()
_NC = _info.num_cores
_NS = _info.num_subcores
_NW = _NC * _NS
_B_PER_W = _B // _NW
_CHUNK = 128
_N_CHUNKS = _B_PER_W // _CHUNK
_NB = 5
_GROUPS = _N_CHUNKS // _NB

_mesh = plsc.VectorSubcoreMesh(core_axis_name="c", subcore_axis_name="s")


@functools.partial(
    pl.kernel,
    mesh=_mesh,
    out_type=jax.ShapeDtypeStruct((_B, _D), jnp.float32),
    scratch_types=[
        pltpu.VMEM((_N_CHUNKS, _CHUNK), jnp.int32),
        pltpu.VMEM((_NB, _CHUNK, _D), jnp.float32),
    ]
    + [pltpu.SemaphoreType.DMA] * (2 * _NB),
)
def _sc_gather(idx_hbm, table_hbm, out_hbm, idx_v, rows_v, *sems):
    gsems = sems[:_NB]
    ssems = sems[_NB:]
    wid = lax.axis_index("s") * _NC + lax.axis_index("c")
    base = wid * _B_PER_W
    pltpu.sync_copy(idx_hbm.at[wid], idx_v)

    for b in range(_NB):
        pltpu.async_copy(table_hbm.at[idx_v.at[b]], rows_v.at[b], gsems[b])

    def group(g, carry):
        for b in range(_NB):
            j = g * _NB + b
            pltpu.make_async_copy(
                table_hbm.at[idx_v.at[j]], rows_v.at[b], gsems[b]
            ).wait()
            pltpu.async_copy(
                rows_v.at[b],
                out_hbm.at[pl.ds(base + j * _CHUNK, _CHUNK)],
                ssems[b],
            )

        @pl.when(g < _GROUPS - 1)
        def _():
            for b in range(_NB):
                jn = (g + 1) * _NB + b
                pltpu.make_async_copy(
                    rows_v.at[b], out_hbm.at[pl.ds(base, _CHUNK)], ssems[b]
                ).wait()
                pltpu.async_copy(
                    table_hbm.at[idx_v.at[jn]], rows_v.at[b], gsems[b]
                )

        return carry

    lax.fori_loop(0, _GROUPS, group, 0)

    for b in range(_NB):
        pltpu.make_async_copy(
            rows_v.at[b], out_hbm.at[pl.ds(base, _CHUNK)], ssems[b]
        ).wait()


def kernel(x, embedding_weights):
    idx = x.reshape(_NW, _N_CHUNKS, _CHUNK)
    out = _sc_gather(idx, embedding_weights)
    return out.reshape(x.shape[0], x.shape[1], _D)


# speedup vs baseline: 5.8923x; 1.7775x over previous
import functools

import jax
import jax.numpy as jnp
from jax import lax
from jax.experimental import pallas as pl
from jax.experimental.pallas import tpu as pltpu
from jax.experimental.pallas import tpu_sc as plsc

_VOCAB = 100000
_D = 128
_ROWS = 4096
_SEQ = 50

_info = plsc.get_sparse_core_info()
_NC = _info.num_cores
_NS = _info.num_subcores
_NW = _NC * _NS
_RPW = _ROWS // _NW
_BLK = 2
_CHUNK = _BLK * _SEQ
_N_CHUNKS = _RPW // _BLK
_NB = 4
_GROUPS = _N_CHUNKS // _NB

_mesh = plsc.VectorSubcoreMesh(core_axis_name="c", subcore_axis_name="s")


@functools.partial(
    pl.kernel,
    mesh=_mesh,
    out_type=jax.ShapeDtypeStruct((_ROWS, _SEQ, _D), jnp.float32),
    scratch_types=[
        pltpu.VMEM((_N_CHUNKS, _CHUNK), jnp.int32),
        pltpu.VMEM((_NB, _CHUNK, _D), jnp.float32),
    ]
    + [pltpu.SemaphoreType.DMA] * (2 * _NB),
    compiler_params=pltpu.CompilerParams(use_tc_tiling_on_sc=True),
)
def _sc_gather(idx_hbm, table_hbm, out_hbm, idx_v, rows_v, *sems):
    gsems = sems[:_NB]
    ssems = sems[_NB:]
    wid = lax.axis_index("s") * _NC + lax.axis_index("c")
    i_base = wid * _RPW
    pltpu.sync_copy(idx_hbm.at[wid], idx_v)

    def scatter_pair(b, j, start):
        i0 = i_base + j * _BLK
        for k in range(_BLK):
            d = pltpu.make_async_copy(
                rows_v.at[b, pl.ds(k * _SEQ, _SEQ)],
                out_hbm.at[i0 + k],
                ssems[b],
            )
            if start:
                d.start()
            else:
                d.wait()

    for b in range(_NB):
        pltpu.async_copy(table_hbm.at[idx_v.at[b]], rows_v.at[b], gsems[b])

    def group(g, carry):
        for b in range(_NB):
            j = g * _NB + b
            pltpu.make_async_copy(
                table_hbm.at[idx_v.at[j]], rows_v.at[b], gsems[b]
            ).wait()
            scatter_pair(b, j, start=True)

        @pl.when(g < _GROUPS - 1)
        def _():
            for b in range(_NB):
                jn = (g + 1) * _NB + b
                scatter_pair(b, jn, start=False)
                pltpu.async_copy(
                    table_hbm.at[idx_v.at[jn]], rows_v.at[b], gsems[b]
                )

        return carry

    lax.fori_loop(0, _GROUPS, group, 0)

    for b in range(_NB):
        scatter_pair(b, _N_CHUNKS - 1, start=False)


def kernel(x, embedding_weights):
    idx = x.reshape(_NW, _N_CHUNKS, _CHUNK)
    return _sc_gather(idx, embedding_weights)


# speedup vs baseline: 10.3904x; 1.7634x over previous
import functools

import jax
import jax.numpy as jnp
from jax import lax
from jax.experimental import pallas as pl
from jax.experimental.pallas import tpu as pltpu
from jax.experimental.pallas import tpu_sc as plsc

_VOCAB = 100000
_D = 128
_ROWS = 4096
_SEQ = 50

_info = plsc.get_sparse_core_info()
_NC = _info.num_cores
_NS = _info.num_subcores
_NW = _NC * _NS
_RPW = _ROWS // _NW
_N_CHUNKS = _SEQ
_NB = 5
_GROUPS = _N_CHUNKS // _NB

_mesh = plsc.VectorSubcoreMesh(core_axis_name="c", subcore_axis_name="s")


@functools.partial(
    pl.kernel,
    mesh=_mesh,
    out_type=jax.ShapeDtypeStruct((_SEQ, _ROWS, _D), jnp.float32),
    scratch_types=[
        pltpu.VMEM((_N_CHUNKS, _RPW), jnp.int32),
        pltpu.VMEM((_NB, _RPW, _D), jnp.float32),
    ]
    + [pltpu.SemaphoreType.DMA] * (2 * _NB),
)
def _sc_gather(idx_hbm, table_hbm, out_hbm, idx_v, rows_v, *sems):
    gsems = sems[:_NB]
    ssems = sems[_NB:]
    wid = lax.axis_index("s") * _NC + lax.axis_index("c")
    i_base = wid * _RPW
    pltpu.sync_copy(idx_hbm.at[wid], idx_v)

    for b in range(_NB):
        pltpu.async_copy(table_hbm.at[idx_v.at[b]], rows_v.at[b], gsems[b])

    def group(g, carry):
        for b in range(_NB):
            j = g * _NB + b
            pltpu.make_async_copy(
                table_hbm.at[idx_v.at[j]], rows_v.at[b], gsems[b]
            ).wait()
            pltpu.async_copy(
                rows_v.at[b],
                out_hbm.at[j, pl.ds(i_base, _RPW)],
                ssems[b],
            )

        @pl.when(g < _GROUPS - 1)
        def _():
            for b in range(_NB):
                jn = (g + 1) * _NB + b
                pltpu.make_async_copy(
                    rows_v.at[b], out_hbm.at[0, pl.ds(i_base, _RPW)], ssems[b]
                ).wait()
                pltpu.async_copy(
                    table_hbm.at[idx_v.at[jn]], rows_v.at[b], gsems[b]
                )

        return carry

    lax.fori_loop(0, _GROUPS, group, 0)

    for b in range(_NB):
        pltpu.make_async_copy(
            rows_v.at[b], out_hbm.at[0, pl.ds(i_base, _RPW)], ssems[b]
        ).wait()


def kernel(x, embedding_weights):
    idx = x.reshape(_NW, _RPW, _SEQ).transpose(0, 2, 1)
    out = _sc_gather(idx, embedding_weights)
    return out.transpose(1, 0, 2)
